# baseline (speedup 1.0000x reference)
import functools

import jax
import jax.numpy as jnp
from jax import lax
from jax.experimental import pallas as pl
from jax.experimental.pallas import tpu as pltpu
from jax.experimental.pallas import tpu_sc as plsc

_NW = 32
_VOCAB = 1000000
_DIM = 32
_HIST = 50
_BATCH = 16384

_mesh = functools.partial(
    plsc.VectorSubcoreMesh, core_axis_name="c", subcore_axis_name="s"
)

_VC = 1024
_N_FULL = _VOCAB // _VC
_TAIL = _VOCAB - _N_FULL * _VC
_A_TPW = -(-_N_FULL // _NW)


def _stage_a(table4):
    @functools.partial(
        pl.kernel,
        mesh=_mesh(),
        out_type=jax.ShapeDtypeStruct((_VOCAB * _DIM,), jnp.float32),
        scratch_types=[
            pltpu.VMEM((4, 8, _VC), jnp.float32),
            pltpu.VMEM((_VC * _DIM,), jnp.float32),
            pltpu.VMEM((4, 8, _TAIL), jnp.float32),
            pltpu.VMEM((_TAIL * _DIM,), jnp.float32),
        ],
        compiler_params=pltpu.CompilerParams(use_tc_tiling_on_sc=True, needs_layout_passes=False),
    )
    def ka(tab_hbm, flat_hbm, in_v, out_v, in_t, out_t):
        c = lax.axis_index("c")
        s = lax.axis_index("s")
        w = s * 2 + c
        lane = lax.iota(jnp.int32, 16)

        lane32 = lane * _DIM

        def task(t, carry):
            @pl.when(t < _N_FULL)
            def _run():
                v0 = t * _VC
                for g in range(4):
                    pltpu.sync_copy(
                        tab_hbm.at[g, :, pl.ds(v0, _VC)], in_v.at[g]
                    )
                for g in range(4):
                    for r in range(8):
                        d = 8 * g + r

                        def body(vb, idxv, g=g, r=r):
                            for u in range(8):
                                x = in_v[g, r, pl.ds(vb * 128 + u * 16, 16)]
                                plsc.store_scatter(
                                    out_v, [idxv + (u * 16 * _DIM)], x
                                )
                            return idxv + (128 * _DIM)

                        lax.fori_loop(0, _VC // 128, body, lane32 + d)
                pltpu.sync_copy(out_v, flat_hbm.at[pl.ds(v0 * _DIM, _VC * _DIM)])

            return carry

        lax.fori_loop(0, _A_TPW, lambda i, cy: task(w + _NW * i, cy), 0)

        @pl.when(w == _NW - 1)
        def _tail_task():
            v0 = _N_FULL * _VC
            for g in range(4):
                pltpu.sync_copy(tab_hbm.at[g, :, pl.ds(v0, _TAIL)], in_t.at[g])
            for g in range(4):
                for r in range(8):
                    d = 8 * g + r

                    def body(vb, idxv, g=g, r=r):
                        x = in_t[g, r, pl.ds(vb * 16, 16)]
                        plsc.store_scatter(out_t, [idxv], x)
                        return idxv + (16 * _DIM)

                    lax.fori_loop(0, _TAIL // 16, body, lane32 + d)
            pltpu.sync_copy(out_t, flat_hbm.at[pl.ds(v0 * _DIM, _TAIL * _DIM)])

    return ka(table4)


_CHUNK = 1024
_B_PER_W = (_BATCH * _HIST) // _NW
_N_CHUNKS = _B_PER_W // _CHUNK


def _stage_b(idx_flat, table_rm):
    total = idx_flat.shape[0]

    @functools.partial(
        pl.kernel,
        mesh=_mesh(),
        out_type=jax.ShapeDtypeStruct((total, _DIM), jnp.float32),
        scratch_types=[
            pltpu.VMEM((_B_PER_W,), jnp.int32),
            pltpu.VMEM((2, _CHUNK, _DIM), jnp.float32),
            pltpu.SemaphoreType.DMA,
            pltpu.SemaphoreType.DMA,
            pltpu.SemaphoreType.DMA,
            pltpu.SemaphoreType.DMA,
        ],
        compiler_params=pltpu.CompilerParams(use_tc_tiling_on_sc=False),
    )
    def kb(idx_hbm, tab_hbm, out_hbm, idx_v, rows_v, sg0, sg1, ss0, ss1):
        w = lax.axis_index("s") * 2 + lax.axis_index("c")
        base = w * _B_PER_W
        pltpu.sync_copy(idx_hbm.at[pl.ds(base, _B_PER_W)], idx_v)

        sem_g = (sg0, sg1)
        sem_s = (ss0, ss1)
        gathers = [None] * _N_CHUNKS
        stores = [None] * _N_CHUNKS
        for i in range(_N_CHUNKS):
            b = i % 2
            if i >= 2:
                stores[i - 2].wait()
            gathers[i] = pltpu.make_async_copy(
                tab_hbm.at[idx_v.at[pl.ds(i * _CHUNK, _CHUNK)]],
                rows_v.at[b],
                sem_g[b],
            )
            gathers[i].start()
            if i >= 1:
                gathers[i - 1].wait()
                stores[i - 1] = pltpu.make_async_copy(
                    rows_v.at[1 - b],
                    out_hbm.at[pl.ds(base + (i - 1) * _CHUNK, _CHUNK)],
                    sem_s[1 - b],
                )
                stores[i - 1].start()
        last = _N_CHUNKS - 1
        gathers[last].wait()
        stores[last] = pltpu.make_async_copy(
            rows_v.at[last % 2],
            out_hbm.at[pl.ds(base + last * _CHUNK, _CHUNK)],
            sem_s[last % 2],
        )
        stores[last].start()
        stores[last - 1].wait()
        stores[last].wait()

    return kb(idx_flat, table_rm)


_BC = 2048
_NTASK_C = _HIST * (_BATCH // _BC)
_C_TPW = -(-_NTASK_C // _NW)


def _stage_c(flat_rows):
    @functools.partial(
        pl.kernel,
        mesh=_mesh(),
        out_type=jax.ShapeDtypeStruct((_HIST, 4, 8, _BATCH), jnp.float32),
        scratch_types=[
            pltpu.VMEM((_BC * _DIM,), jnp.float32),
            pltpu.VMEM((_BC,), jnp.float32),
            pltpu.SemaphoreType.DMA,
        ],
        compiler_params=pltpu.CompilerParams(use_tc_tiling_on_sc=True, needs_layout_passes=False),
    )
    def kc(rows_hbm, out_hbm, in_v, vals_v, sem):
        c = lax.axis_index("c")
        s = lax.axis_index("s")
        w = s * 2 + c
        lane = lax.iota(jnp.int32, 16)
        nb = _BATCH // _BC

        def task(t, carry):
            @pl.when(t < _NTASK_C)
            def _run():
                h = t // nb
                bq = t % nb
                pltpu.sync_copy(
                    rows_hbm.at[pl.ds((h * _BATCH + bq * _BC) * _DIM, _BC * _DIM)],
                    in_v,
                )
                lane32 = lane * _DIM
                for g in range(4):
                    for r in range(8):
                        d = 8 * g + r

                        def body(vb, idxv, g=g, r=r):
                            for u in range(8):
                                x = plsc.load_gather(
                                    in_v, [idxv + (u * 16 * _DIM)]
                                )
                                vals_v[pl.ds(vb * 128 + u * 16, 16)] = x
                            return idxv + (128 * _DIM)

                        lax.fori_loop(0, _BC // 128, body, lane32 + d)
                        pltpu.sync_copy(
                            vals_v, out_hbm.at[h, g, r, pl.ds(bq * _BC, _BC)]
                        )

            return carry

        lax.fori_loop(0, _C_TPW, lambda i, cy: task(w + _NW * i, cy), 0)

    return kc(flat_rows)


@jax.jit
def _embed(idx_flat, table4):
    table_flat = _stage_a(table4)
    table_rm = table_flat.reshape(_VOCAB, _DIM)
    rows = _stage_b(idx_flat, table_rm)
    out_t = _stage_c(rows.reshape(-1))
    return out_t


def kernel(input_ids, table):
    idx_flat = input_ids.T.astype(jnp.int32).reshape(-1)
    table4 = table.T.reshape(4, 8, _VOCAB)
    out_t = _embed(idx_flat, table4)
    out = out_t.reshape(_HIST, _DIM, _BATCH)
    return jnp.transpose(out, (2, 0, 1))


# speedup vs baseline: 1.2781x; 1.2781x over previous
import functools

import jax
import jax.numpy as jnp
from jax import lax
from jax.experimental import pallas as pl
from jax.experimental.pallas import tpu as pltpu
from jax.experimental.pallas import tpu_sc as plsc

_NW = 32
_VOCAB = 1000000
_DIM = 32
_HIST = 50
_BATCH = 16384

_mesh = functools.partial(
    plsc.VectorSubcoreMesh, core_axis_name="c", subcore_axis_name="s"
)

_VPB = 2048
_A_GRID = -(-_VOCAB // _VPB)
_A_ROWS = _A_GRID * _VPB


def _stage_a(table_t):
    def ka(in_ref, out_ref):
        x = in_ref[...]
        out_ref[...] = jnp.concatenate(
            [x[:, 512 * j : 512 * (j + 1)].T for j in range(4)], axis=1
        )

    return pl.pallas_call(
        ka,
        grid=(_A_GRID,),
        in_specs=[pl.BlockSpec((_DIM, _VPB), lambda i: (0, i))],
        out_specs=pl.BlockSpec((_VPB // 4, 128), lambda i: (i, 0)),
        out_shape=jax.ShapeDtypeStruct((_A_ROWS * _DIM // 128, 128), jnp.float32),
    )(table_t)


_CHUNK = 1024
_B_PER_W = (_BATCH * _HIST) // _NW
_N_CHUNKS = _B_PER_W // _CHUNK


def _stage_b(idx_flat, table_rm):
    total = idx_flat.shape[0]

    @functools.partial(
        pl.kernel,
        mesh=_mesh(),
        out_type=jax.ShapeDtypeStruct((total, _DIM), jnp.float32),
        scratch_types=[
            pltpu.VMEM((_B_PER_W,), jnp.int32),
            pltpu.VMEM((2, _CHUNK, _DIM), jnp.float32),
            pltpu.SemaphoreType.DMA,
            pltpu.SemaphoreType.DMA,
            pltpu.SemaphoreType.DMA,
            pltpu.SemaphoreType.DMA,
        ],
        compiler_params=pltpu.CompilerParams(use_tc_tiling_on_sc=False),
    )
    def kb(idx_hbm, tab_hbm, out_hbm, idx_v, rows_v, sg0, sg1, ss0, ss1):
        w = lax.axis_index("s") * 2 + lax.axis_index("c")
        base = w * _B_PER_W
        pltpu.sync_copy(idx_hbm.at[pl.ds(base, _B_PER_W)], idx_v)

        sem_g = (sg0, sg1)
        sem_s = (ss0, ss1)
        gathers = [None] * _N_CHUNKS
        stores = [None] * _N_CHUNKS
        for i in range(_N_CHUNKS):
            b = i % 2
            if i >= 2:
                stores[i - 2].wait()
            gathers[i] = pltpu.make_async_copy(
                tab_hbm.at[idx_v.at[pl.ds(i * _CHUNK, _CHUNK)]],
                rows_v.at[b],
                sem_g[b],
            )
            gathers[i].start()
            if i >= 1:
                gathers[i - 1].wait()
                stores[i - 1] = pltpu.make_async_copy(
                    rows_v.at[1 - b],
                    out_hbm.at[pl.ds(base + (i - 1) * _CHUNK, _CHUNK)],
                    sem_s[1 - b],
                )
                stores[i - 1].start()
        last = _N_CHUNKS - 1
        gathers[last].wait()
        stores[last] = pltpu.make_async_copy(
            rows_v.at[last % 2],
            out_hbm.at[pl.ds(base + last * _CHUNK, _CHUNK)],
            sem_s[last % 2],
        )
        stores[last].start()
        stores[last - 1].wait()
        stores[last].wait()

    return kb(idx_flat, table_rm)


def _stage_c(rows128):
    def kc(in_ref, out_ref):
        x = in_ref[...]
        cat = jnp.concatenate(
            [x[:, 32 * j : 32 * (j + 1)].T for j in range(4)], axis=1
        )
        out_ref[...] = cat.reshape(1, _DIM, 8, 128)

    return pl.pallas_call(
        kc,
        grid=(_HIST, _BATCH // 1024),
        in_specs=[pl.BlockSpec((256, 128), lambda h, c: (h * 16 + c, 0))],
        out_specs=pl.BlockSpec((1, _DIM, 8, 128), lambda h, c: (h, 0, c, 0)),
        out_shape=jax.ShapeDtypeStruct(
            (_HIST, _DIM, _BATCH // 128, 128), jnp.float32
        ),
    )(rows128)


@jax.jit
def _embed(idx_flat, table_t):
    table_p = _stage_a(table_t)
    table_rm = table_p.reshape(_A_ROWS, _DIM)
    rows = _stage_b(idx_flat, table_rm)
    out_t = _stage_c(rows.reshape(-1, 128))
    return out_t


def kernel(input_ids, table):
    idx = input_ids.T.astype(jnp.int32)
    idx2 = (
        idx.reshape(_HIST, _BATCH // 1024, 4, 256)
        .transpose(0, 1, 3, 2)
        .reshape(-1)
    )
    idx2 = (idx2 & ~2047) | ((idx2 & 511) << 2) | ((idx2 >> 9) & 3)
    out_t = _embed(idx2, table.T)
    out = out_t.reshape(_HIST, _DIM, _BATCH)
    return jnp.transpose(out, (2, 0, 1))


# speedup vs baseline: 1.7494x; 1.3687x over previous
import functools

import jax
import jax.numpy as jnp
from jax import lax
from jax.experimental import pallas as pl
from jax.experimental.pallas import tpu as pltpu
from jax.experimental.pallas import tpu_sc as plsc

_NW = 32
_VOCAB = 1000000
_DIM = 32
_HIST = 50
_BATCH = 16384

_mesh = functools.partial(
    plsc.VectorSubcoreMesh, core_axis_name="c", subcore_axis_name="s"
)

_VPB = 4096
_A_GRID = -(-_VOCAB // _VPB)
_A_ROWS = _A_GRID * _VPB


def _stage_a(table_t):
    def ka(in_ref, out_ref):
        x = in_ref[...]
        eye = jnp.eye(_DIM, dtype=jnp.float32)
        out_ref[...] = jnp.concatenate(
            [
                jnp.concatenate(
                    [
                        lax.dot_general(
                            x[:, 2048 * g + 512 * j : 2048 * g + 512 * (j + 1)],
                            eye,
                            (((0,), (0,)), ((), ())),
                        )
                        for j in range(4)
                    ],
                    axis=1,
                )
                for g in range(2)
            ],
            axis=0,
        )

    return pl.pallas_call(
        ka,
        grid=(_A_GRID,),
        in_specs=[pl.BlockSpec((_DIM, _VPB), lambda i: (0, i))],
        out_specs=pl.BlockSpec((_VPB // 4, 128), lambda i: (i, 0)),
        out_shape=jax.ShapeDtypeStruct((_A_ROWS * _DIM // 128, 128), jnp.float32),
    )(table_t)


_CHUNK = 1024
_B_PER_W = (_BATCH * _HIST) // _NW
_N_CHUNKS = _B_PER_W // _CHUNK


def _stage_b(idx_flat, table_rm):
    total = idx_flat.shape[0]

    @functools.partial(
        pl.kernel,
        mesh=_mesh(),
        out_type=jax.ShapeDtypeStruct((total, _DIM), jnp.float32),
        scratch_types=[
            pltpu.VMEM((_B_PER_W,), jnp.int32),
            pltpu.VMEM((2, _CHUNK, _DIM), jnp.float32),
            pltpu.SemaphoreType.DMA,
            pltpu.SemaphoreType.DMA,
            pltpu.SemaphoreType.DMA,
            pltpu.SemaphoreType.DMA,
        ],
        compiler_params=pltpu.CompilerParams(use_tc_tiling_on_sc=False),
    )
    def kb(idx_hbm, tab_hbm, out_hbm, idx_v, rows_v, sg0, sg1, ss0, ss1):
        w = lax.axis_index("s") * 2 + lax.axis_index("c")
        base = w * _B_PER_W
        pltpu.sync_copy(idx_hbm.at[pl.ds(base, _B_PER_W)], idx_v)

        sem_g = (sg0, sg1)
        sem_s = (ss0, ss1)
        gathers = [None] * _N_CHUNKS
        stores = [None] * _N_CHUNKS
        for i in range(_N_CHUNKS):
            b = i % 2
            if i >= 2:
                stores[i - 2].wait()
            gathers[i] = pltpu.make_async_copy(
                tab_hbm.at[idx_v.at[pl.ds(i * _CHUNK, _CHUNK)]],
                rows_v.at[b],
                sem_g[b],
            )
            gathers[i].start()
            if i >= 1:
                gathers[i - 1].wait()
                stores[i - 1] = pltpu.make_async_copy(
                    rows_v.at[1 - b],
                    out_hbm.at[pl.ds(base + (i - 1) * _CHUNK, _CHUNK)],
                    sem_s[1 - b],
                )
                stores[i - 1].start()
        last = _N_CHUNKS - 1
        gathers[last].wait()
        stores[last] = pltpu.make_async_copy(
            rows_v.at[last % 2],
            out_hbm.at[pl.ds(base + last * _CHUNK, _CHUNK)],
            sem_s[last % 2],
        )
        stores[last].start()
        stores[last - 1].wait()
        stores[last].wait()

    return kb(idx_flat, table_rm)


def _stage_c(rows128):
    def kc(in_ref, out_ref):
        x = in_ref[...]
        eye = jnp.eye(_DIM, dtype=jnp.float32)
        cat = jnp.concatenate(
            [
                lax.dot_general(
                    eye,
                    x[256 * g : 256 * (g + 1), 32 * j : 32 * (j + 1)],
                    (((1,), (1,)), ((), ())),
                )
                for g in range(2)
                for j in range(4)
            ],
            axis=1,
        )
        out_ref[...] = cat.reshape(1, _DIM, 16, 128)

    return pl.pallas_call(
        kc,
        grid=(_HIST, _BATCH // 2048),
        in_specs=[pl.BlockSpec((512, 128), lambda h, c: (h * 8 + c, 0))],
        out_specs=pl.BlockSpec((1, _DIM, 16, 128), lambda h, c: (h, 0, c, 0)),
        out_shape=jax.ShapeDtypeStruct(
            (_HIST, _DIM, _BATCH // 128, 128), jnp.float32
        ),
    )(rows128)


@jax.jit
def _embed(idx_flat, table_t):
    table_p = _stage_a(table_t)
    table_rm = table_p.reshape(_A_ROWS, _DIM)
    rows = _stage_b(idx_flat, table_rm)
    out_t = _stage_c(rows.reshape(-1, 128))
    return out_t


def kernel(input_ids, table):
    idx = input_ids.T.astype(jnp.int32)
    idx2 = (
        idx.reshape(_HIST, _BATCH // 1024, 4, 256)
        .transpose(0, 1, 3, 2)
        .reshape(-1)
    )
    idx2 = (idx2 & ~2047) | ((idx2 & 511) << 2) | ((idx2 >> 9) & 3)
    out_t = _embed(idx2, table.T)
    out = out_t.reshape(_HIST, _DIM, _BATCH)
    return jnp.transpose(out, (2, 0, 1))


# speedup vs baseline: 2.0924x; 1.1961x over previous
import functools

import jax
import jax.numpy as jnp
from jax import lax
from jax.experimental import pallas as pl
from jax.experimental.pallas import tpu as pltpu
from jax.experimental.pallas import tpu_sc as plsc

_NW = 32
_VOCAB = 1000000
_DIM = 32
_HIST = 50
_BATCH = 16384

_mesh = functools.partial(
    plsc.VectorSubcoreMesh, core_axis_name="c", subcore_axis_name="s"
)

_VPB = 8192
_A_GRID = -(-_VOCAB // _VPB)
_A_ROWS = _A_GRID * _VPB


def _stage_a(table_t):
    def ka(in_ref, out_ref):
        x = in_ref[...]
        eye = jnp.eye(_DIM, dtype=jnp.float32)
        out_ref[...] = jnp.concatenate(
            [
                jnp.concatenate(
                    [
                        lax.dot_general(
                            x[:, 2048 * g + 512 * j : 2048 * g + 512 * (j + 1)],
                            eye,
                            (((0,), (0,)), ((), ())),
                        )
                        for j in range(4)
                    ],
                    axis=1,
                )
                for g in range(4)
            ],
            axis=0,
        )

    return pl.pallas_call(
        ka,
        grid=(_A_GRID,),
        in_specs=[pl.BlockSpec((_DIM, _VPB), lambda i: (0, i))],
        out_specs=pl.BlockSpec((_VPB // 4, 128), lambda i: (i, 0)),
        out_shape=jax.ShapeDtypeStruct((_A_ROWS * _DIM // 128, 128), jnp.float32),
    )(table_t)


_CHUNK = 1024
_B_PER_W = (_BATCH * _HIST) // _NW
_N_CHUNKS = _B_PER_W // _CHUNK


def _stage_b(idx_flat, table_rm):
    total = idx_flat.shape[0]

    @functools.partial(
        pl.kernel,
        mesh=_mesh(),
        out_type=jax.ShapeDtypeStruct((total, _DIM), jnp.float32),
        scratch_types=[
            pltpu.VMEM((_B_PER_W,), jnp.int32),
            pltpu.VMEM((2, _CHUNK, _DIM), jnp.float32),
            pltpu.SemaphoreType.DMA,
            pltpu.SemaphoreType.DMA,
            pltpu.SemaphoreType.DMA,
            pltpu.SemaphoreType.DMA,
        ],
        compiler_params=pltpu.CompilerParams(use_tc_tiling_on_sc=False),
    )
    def kb(idx_hbm, tab_hbm, out_hbm, idx_v, rows_v, sg0, sg1, ss0, ss1):
        w = lax.axis_index("s") * 2 + lax.axis_index("c")
        base = w * _B_PER_W
        pltpu.sync_copy(idx_hbm.at[pl.ds(base, _B_PER_W)], idx_v)

        sem_g = (sg0, sg1)
        sem_s = (ss0, ss1)
        gathers = [None] * _N_CHUNKS
        stores = [None] * _N_CHUNKS
        for i in range(_N_CHUNKS):
            b = i % 2
            if i >= 2:
                stores[i - 2].wait()
            gathers[i] = pltpu.make_async_copy(
                tab_hbm.at[idx_v.at[pl.ds(i * _CHUNK, _CHUNK)]],
                rows_v.at[b],
                sem_g[b],
            )
            gathers[i].start()
            if i >= 1:
                gathers[i - 1].wait()
                stores[i - 1] = pltpu.make_async_copy(
                    rows_v.at[1 - b],
                    out_hbm.at[pl.ds(base + (i - 1) * _CHUNK, _CHUNK)],
                    sem_s[1 - b],
                )
                stores[i - 1].start()
        last = _N_CHUNKS - 1
        gathers[last].wait()
        stores[last] = pltpu.make_async_copy(
            rows_v.at[last % 2],
            out_hbm.at[pl.ds(base + last * _CHUNK, _CHUNK)],
            sem_s[last % 2],
        )
        stores[last].start()
        stores[last - 1].wait()
        stores[last].wait()

    return kb(idx_flat, table_rm)


def _stage_c(rows128):
    def kc(in_ref, out_ref):
        x = in_ref[...]
        eye = jnp.eye(_DIM, dtype=jnp.float32)
        cat = jnp.concatenate(
            [
                lax.dot_general(
                    eye,
                    x[256 * g : 256 * (g + 1), 32 * j : 32 * (j + 1)],
                    (((1,), (1,)), ((), ())),
                )
                for g in range(4)
                for j in range(4)
            ],
            axis=1,
        )
        out_ref[...] = cat.reshape(1, _DIM, 32, 128)

    return pl.pallas_call(
        kc,
        grid=(_HIST, _BATCH // 4096),
        in_specs=[pl.BlockSpec((1024, 128), lambda h, c: (h * 4 + c, 0))],
        out_specs=pl.BlockSpec((1, _DIM, 32, 128), lambda h, c: (h, 0, c, 0)),
        out_shape=jax.ShapeDtypeStruct(
            (_HIST, _DIM, _BATCH // 128, 128), jnp.float32
        ),
    )(rows128)


@jax.jit
def _embed(idx_flat, table_t):
    table_p = _stage_a(table_t)
    table_rm = table_p.reshape(_A_ROWS, _DIM)
    rows = _stage_b(idx_flat, table_rm)
    out_t = _stage_c(rows.reshape(-1, 128))
    return out_t


def kernel(input_ids, table):
    idx = input_ids.T.astype(jnp.int32)
    idx2 = (
        idx.reshape(_HIST, _BATCH // 1024, 4, 256)
        .transpose(0, 1, 3, 2)
        .reshape(-1)
    )
    idx2 = (idx2 & ~2047) | ((idx2 & 511) << 2) | ((idx2 >> 9) & 3)
    out_t = _embed(idx2, table.T)
    out = out_t.reshape(_HIST, _DIM, _BATCH)
    return jnp.transpose(out, (2, 0, 1))
